# Initial kernel scaffold; baseline (speedup 1.0000x reference)
import jax
import jax.numpy as jnp
from jax.experimental import pallas as pl


def kernel(x, edge_index, batch, W_rel1, b_rel1, W_root1, W_rel2, b_rel2, W_root2, W_rel3, b_rel3, W_root3, bn_gamma, bn_beta, W_lin, b_lin):
    raise NotImplementedError("write your pallas kernel here")


# speedup vs baseline: 7.2878x; 7.2878x over previous
import functools

import jax
import jax.numpy as jnp
from jax import lax
from jax.experimental import pallas as pl
from jax.experimental.pallas import tpu as pltpu
from jax.experimental.pallas import tpu_sc as plsc

N = 10000
E = 320000
D = 128
H = 128
C = 32
G = 64

NC = 2
NS = 16
NW = NC * NS
EPT = E // NW
BLK = 125
NBLK = EPT // BLK
NPAD = 10240
RPT = NPAD // NS

_mesh = plsc.VectorSubcoreMesh(core_axis_name="c", subcore_axis_name="s")


def _segment_sum_sc(h, src3, dst3, zblk):
    @functools.partial(
        pl.kernel,
        out_type=jax.ShapeDtypeStruct((NC * NPAD, H), jnp.float32),
        mesh=_mesh,
        scratch_types=[
            pltpu.VMEM((NBLK, BLK), jnp.int32),
            pltpu.VMEM((NBLK, BLK), jnp.int32),
            pltpu.VMEM((BLK, H), jnp.float32),
            pltpu.VMEM_SHARED((NPAD, H), jnp.float32),
        ],
    )
    def k(h_hbm, src_hbm, dst_hbm, z_hbm, out_hbm, src_v, dst_v, rows_v, acc):
        c = lax.axis_index("c")
        s = lax.axis_index("s")
        tid = c * NS + s
        pltpu.sync_copy(z_hbm, acc.at[pl.ds(s * RPT, RPT)])
        pltpu.sync_copy(src_hbm.at[tid], src_v)
        pltpu.sync_copy(dst_hbm.at[tid], dst_v)
        plsc.subcore_barrier()
        @pl.loop(0, NBLK)
        def _(j):
            pltpu.sync_copy(h_hbm.at[src_v.at[j]], rows_v)
            pltpu.sync_copy(rows_v, acc.at[dst_v.at[j]], add=True)
        plsc.subcore_barrier()
        pltpu.sync_copy(acc.at[pl.ds(s * RPT, RPT)],
                        out_hbm.at[pl.ds(c * NPAD + s * RPT, RPT)])

    return k(h, src3, dst3, zblk)


def _dense_body(p_ref, h_ref, wr_ref, br_ref, wt_ref, g_ref, b_ref, o_ref):
    agg = p_ref[0, :N] + p_ref[1, :N]
    y = (lax.dot(agg, wr_ref[...], precision=lax.Precision.HIGHEST,
                 preferred_element_type=jnp.float32)
         + lax.dot(h_ref[...], wt_ref[...], precision=lax.Precision.HIGHEST,
                   preferred_element_type=jnp.float32)
         + br_ref[...])
    y = jnp.maximum(y, 0.0)
    mean = jnp.mean(y, axis=0, keepdims=True)
    var = jnp.mean((y - mean) ** 2, axis=0, keepdims=True)
    o_ref[...] = (y - mean) * lax.rsqrt(var + 1e-5) * g_ref[...] + b_ref[...]


def _dense_tc(parts, h, wr, br, wt, gamma, beta):
    parts = parts.reshape(NC, NPAD, H)
    return pl.pallas_call(
        _dense_body,
        out_shape=jax.ShapeDtypeStruct((N, H), jnp.float32),
    )(parts, h, wr, br, wt, gamma, beta)


def _dense_pool_body(p_ref, h_ref, wr_ref, br_ref, wt_ref, g_ref, b_ref,
                     batch_ref, wl_ref, bl_ref, o_ref):
    agg = p_ref[0, :N] + p_ref[1, :N]
    y = (lax.dot(agg, wr_ref[...], precision=lax.Precision.HIGHEST,
                 preferred_element_type=jnp.float32)
         + lax.dot(h_ref[...], wt_ref[...], precision=lax.Precision.HIGHEST,
                   preferred_element_type=jnp.float32)
         + br_ref[...])
    y = jnp.maximum(y, 0.0)
    mean = jnp.mean(y, axis=0, keepdims=True)
    var = jnp.mean((y - mean) ** 2, axis=0, keepdims=True)
    y = (y - mean) * lax.rsqrt(var + 1e-5) * g_ref[...] + b_ref[...]
    onehot = (batch_ref[...] ==
              lax.broadcasted_iota(jnp.int32, (G, N), 0)).astype(jnp.float32)
    sums = lax.dot(onehot, y, precision=lax.Precision.HIGHEST,
                   preferred_element_type=jnp.float32)
    counts = jnp.sum(onehot, axis=1, keepdims=True)
    pooled = sums / jnp.maximum(counts, 1.0)
    o_ref[...] = (lax.dot(pooled, wl_ref[...], precision=lax.Precision.HIGHEST,
                          preferred_element_type=jnp.float32) + bl_ref[...])


def _dense_pool_tc(parts, h, wr, br, wt, gamma, beta, batch2, wl, bl):
    parts = parts.reshape(NC, NPAD, H)
    return pl.pallas_call(
        _dense_pool_body,
        out_shape=jax.ShapeDtypeStruct((G, C), jnp.float32),
    )(parts, h, wr, br, wt, gamma, beta, batch2, wl, bl)


def kernel(x, edge_index, batch, W_rel1, b_rel1, W_root1, W_rel2, b_rel2,
           W_root2, W_rel3, b_rel3, W_root3, bn_gamma, bn_beta, W_lin, b_lin):
    src3 = edge_index[0].reshape(NW, NBLK, BLK)
    dst3 = edge_index[1].reshape(NW, NBLK, BLK)
    zblk = jnp.zeros((RPT, H), jnp.float32)
    batch2 = batch.reshape(1, N)
    g2 = bn_gamma.reshape(1, H)
    bt2 = bn_beta.reshape(1, H)
    bl2 = b_lin.reshape(1, C)

    h = x
    layers = ((W_rel1, b_rel1, W_root1), (W_rel2, b_rel2, W_root2),
              (W_rel3, b_rel3, W_root3))
    for i, (Wr, br, Wt) in enumerate(layers):
        parts = _segment_sum_sc(h, src3, dst3, zblk)
        br2 = br.reshape(1, H)
        if i < 2:
            h = _dense_tc(parts, h, Wr, br2, Wt, g2, bt2)
        else:
            out = _dense_pool_tc(parts, h, Wr, br2, Wt, g2, bt2,
                                 batch2, W_lin, bl2)
    return out
